# Initial kernel scaffold; baseline (speedup 1.0000x reference)
import jax
import jax.numpy as jnp
from jax.experimental import pallas as pl


def kernel(edge_index, edge_weight, emb_weight):
    raise NotImplementedError("write your pallas kernel here")


# speedup vs baseline: 3.3634x; 3.3634x over previous
import functools

import jax
import jax.numpy as jnp
from jax import lax
from jax.experimental import pallas as pl
from jax.experimental.pallas import tpu as pltpu
from jax.experimental.pallas import tpu_sc as plsc

N_USERS = 4000
N_ITEMS = 6000
N = N_USERS + N_ITEMS
DIM = 128
N_LAYERS = 3
E = 320000

NC = 2
NS = 16
NW = NC * NS
CHUNK = 128
CPW = 79
EPW = CPW * CHUNK
EP = NW * EPW
NCH = NW * CPW
NP = 10240
RPT = NP // NS


def _sc_layer_kernel(x_hbm, e_hbm, w_hbm, z_hbm, out_hbm, eb, wv, rows, acc, sem):
    c = lax.axis_index("c")
    s = lax.axis_index("s")
    wid = s * NC + c

    pltpu.sync_copy(z_hbm.at[pl.ds(s * RPT, RPT)], acc.at[pl.ds(s * RPT, RPT)])
    plsc.subcore_barrier()

    cbase = wid * CPW

    def chunk_body(ci, carry):
        pltpu.sync_copy(e_hbm.at[cbase + ci], eb)
        pltpu.sync_copy(w_hbm.at[cbase + ci], wv)
        pltpu.async_copy(x_hbm.at[eb.at[0]], rows, sem).wait()

        def row_body(r, carry2):
            widx = jnp.broadcast_to(r, (16,)).astype(jnp.int32)
            wspl = plsc.load_gather(wv, [widx])
            for j in range(DIM // 16):
                rows[r, pl.ds(j * 16, 16)] = rows[r, pl.ds(j * 16, 16)] * wspl
            return carry2

        lax.fori_loop(0, CHUNK, row_body, 0, unroll=False)

        pltpu.sync_copy(rows, acc.at[eb.at[1]], add=True)
        return carry

    lax.fori_loop(0, CPW, chunk_body, 0, unroll=False)

    plsc.subcore_barrier()
    pltpu.sync_copy(
        acc.at[pl.ds(s * RPT, RPT)],
        out_hbm.at[c, pl.ds(s * RPT, RPT)],
    )


@functools.cache
def _get_sc_layer():
    return functools.partial(
        pl.kernel,
        out_type=jax.ShapeDtypeStruct((NC, NP, DIM), jnp.float32),
        mesh=plsc.VectorSubcoreMesh(
            core_axis_name="c",
            subcore_axis_name="s",
            num_cores=NC,
            num_subcores=NS,
        ),
        scratch_types=[
            pltpu.VMEM((2, CHUNK), jnp.int32),
            pltpu.VMEM((CHUNK,), jnp.float32),
            pltpu.VMEM((CHUNK, DIM), jnp.float32),
            pltpu.VMEM_SHARED((NP, DIM), jnp.float32),
            pltpu.SemaphoreType.DMA,
        ],
        compiler_params=pltpu.CompilerParams(needs_layout_passes=False),
    )(_sc_layer_kernel)


BR = 1000


def _normalize_block(y):
    nrm = jnp.sqrt(jnp.sum(y * y, axis=1, keepdims=True))
    return y / jnp.maximum(nrm, 1e-12)


def _combine_body(p0_ref, p1_ref, s_ref, y_ref, snew_ref):
    y = _normalize_block(p0_ref[0] + p1_ref[0])
    y_ref[...] = y
    snew_ref[...] = s_ref[...] + y


_P_SPECS = [
    pl.BlockSpec((1, BR, DIM), lambda i: (0, i, 0)),
    pl.BlockSpec((1, BR, DIM), lambda i: (1, i, 0)),
    pl.BlockSpec((BR, DIM), lambda i: (i, 0)),
]

_combine = pl.pallas_call(
    _combine_body,
    grid=(N // BR,),
    in_specs=_P_SPECS,
    out_specs=[
        pl.BlockSpec((BR, DIM), lambda i: (i, 0)),
        pl.BlockSpec((BR, DIM), lambda i: (i, 0)),
    ],
    out_shape=[
        jax.ShapeDtypeStruct((N, DIM), jnp.float32),
        jax.ShapeDtypeStruct((N, DIM), jnp.float32),
    ],
)


def _combine_final_body(p0_ref, p1_ref, s_ref, out_ref):
    y = _normalize_block(p0_ref[0] + p1_ref[0])
    out_ref[...] = _normalize_block(s_ref[...] + y)


_combine_final = pl.pallas_call(
    _combine_final_body,
    grid=(N // BR,),
    in_specs=_P_SPECS,
    out_specs=pl.BlockSpec((BR, DIM), lambda i: (i, 0)),
    out_shape=jax.ShapeDtypeStruct((N, DIM), jnp.float32),
)


def kernel(edge_index, edge_weight, emb_weight):
    src = edge_index[0]
    dst = edge_index[1]
    pad = EP - E
    srcp = jnp.concatenate([src, jnp.zeros((pad,), jnp.int32)])
    dstp = jnp.concatenate([dst, jnp.zeros((pad,), jnp.int32)])
    wp = jnp.concatenate([edge_weight, jnp.zeros((pad,), jnp.float32)])
    e_packed = jnp.stack(
        [srcp.reshape(NCH, CHUNK), dstp.reshape(NCH, CHUNK)], axis=1
    )
    w_packed = wp.reshape(NCH, CHUNK)
    zeros = jnp.zeros((NP, DIM), jnp.float32)

    x = emb_weight
    s = emb_weight
    for layer in range(N_LAYERS):
        partials = _get_sc_layer()(x, e_packed, w_packed, zeros)
        if layer < N_LAYERS - 1:
            x, s = _combine(partials, partials, s)
        else:
            out = _combine_final(partials, partials, s)
    return out


# speedup vs baseline: 3.3759x; 1.0037x over previous
import functools

import jax
import jax.numpy as jnp
from jax import lax
from jax.experimental import pallas as pl
from jax.experimental.pallas import tpu as pltpu
from jax.experimental.pallas import tpu_sc as plsc

N_USERS = 4000
N_ITEMS = 6000
N = N_USERS + N_ITEMS
DIM = 128
N_LAYERS = 3
E = 320000

NC = 2
NS = 16
NW = NC * NS
CHUNK = 128
CPW = 80
EPW = CPW * CHUNK
EP = NW * EPW
NP = 10240
RPT = NP // NS


def _sc_layer_kernel(
    x_hbm, sd_hbm, w_hbm, z_hbm, out_hbm,
    sdall, sidx, didx, wv0, wv1, rows, acc,
    gsem0, gsem1, ssem0, ssem1, msem0, msem1,
):
    c = lax.axis_index("c")
    s = lax.axis_index("s")
    wid = s * NC + c
    gsems = (gsem0, gsem1)
    ssems = (ssem0, ssem1)
    msems = (msem0, msem1)
    wvs = (wv0, wv1)

    pltpu.sync_copy(sd_hbm.at[wid], sdall)

    pltpu.sync_copy(z_hbm.at[pl.ds(s * RPT, RPT)], acc.at[pl.ds(s * RPT, RPT)])
    plsc.subcore_barrier()

    def unpack(ci, b):
        for j in range(CHUNK // 16):
            sl = pl.ds(j * 16, 16)
            sd = sdall[ci, sl]
            sidx[b, sl] = sd & 0xFFFF
            didx[b, sl] = lax.shift_right_logical(sd, 16)

    def issue_gather(b):
        pltpu.async_copy(x_hbm.at[sidx.at[b]], rows.at[b], gsems[b])

    def wait_gather(b):
        pltpu.make_async_copy(z_hbm.at[pl.ds(0, CHUNK)], rows.at[b], gsems[b]).wait()

    def issue_scatter(b):
        pltpu.async_copy(rows.at[b], acc.at[didx.at[b]], ssems[b], add=True)

    def wait_scatter(b):
        pltpu.make_async_copy(z_hbm.at[pl.ds(0, CHUNK)], rows.at[b], ssems[b]).wait()

    def issue_wv(ci, b):
        pltpu.async_copy(w_hbm.at[wid, pl.ds(ci * CHUNK, CHUNK)], wvs[b], msems[b])

    def wait_wv(b):
        pltpu.make_async_copy(w_hbm.at[wid, pl.ds(0, CHUNK)], wvs[b], msems[b]).wait()

    def scale(b):
        wv = wvs[b]

        def row_body(r, carry):
            widx = jnp.broadcast_to(r, (16,)).astype(jnp.int32)
            wspl = plsc.load_gather(wv, [widx])
            for j in range(DIM // 16):
                rows[b, r, pl.ds(j * 16, 16)] = rows[b, r, pl.ds(j * 16, 16)] * wspl
            return carry

        lax.fori_loop(0, CHUNK, row_body, 0, unroll=4)

    unpack(0, 0)
    issue_gather(0)
    issue_wv(0, 0)
    unpack(1, 1)
    issue_gather(1)
    issue_wv(1, 1)
    wait_gather(0)
    wait_wv(0)
    scale(0)
    issue_scatter(0)

    def step(ci, p):
        q = 1 - p
        wait_scatter(q)
        unpack(ci + 1, q)
        issue_gather(q)
        issue_wv(ci + 1, q)
        wait_gather(p)
        wait_wv(p)
        scale(p)
        issue_scatter(p)

    def pair_body(g, carry):
        ci = 2 * g + 1
        step(ci, 1)
        step(ci + 1, 0)
        return carry

    lax.fori_loop(0, (CPW - 2) // 2, pair_body, 0, unroll=False)

    wait_gather(1)
    wait_wv(1)
    scale(1)
    issue_scatter(1)
    wait_scatter(0)
    wait_scatter(1)

    plsc.subcore_barrier()
    pltpu.sync_copy(
        acc.at[pl.ds(s * RPT, RPT)],
        out_hbm.at[c, pl.ds(s * RPT, RPT)],
    )


@functools.cache
def _get_sc_layer():
    return functools.partial(
        pl.kernel,
        out_type=jax.ShapeDtypeStruct((NC, NP, DIM), jnp.float32),
        mesh=plsc.VectorSubcoreMesh(
            core_axis_name="c",
            subcore_axis_name="s",
            num_cores=NC,
            num_subcores=NS,
        ),
        scratch_types=[
            pltpu.VMEM((CPW, CHUNK), jnp.int32),
            pltpu.VMEM((2, CHUNK), jnp.int32),
            pltpu.VMEM((2, CHUNK), jnp.int32),
            pltpu.VMEM((CHUNK,), jnp.float32),
            pltpu.VMEM((CHUNK,), jnp.float32),
            pltpu.VMEM((2, CHUNK, DIM), jnp.float32),
            pltpu.VMEM_SHARED((NP, DIM), jnp.float32),
            pltpu.SemaphoreType.DMA,
            pltpu.SemaphoreType.DMA,
            pltpu.SemaphoreType.DMA,
            pltpu.SemaphoreType.DMA,
            pltpu.SemaphoreType.DMA,
            pltpu.SemaphoreType.DMA,
        ],
        compiler_params=pltpu.CompilerParams(needs_layout_passes=False),
    )(_sc_layer_kernel)


BR = 1000


def _normalize_block(y):
    nrm = jnp.sqrt(jnp.sum(y * y, axis=1, keepdims=True))
    return y / jnp.maximum(nrm, 1e-12)


def _combine_body(p0_ref, p1_ref, s_ref, y_ref, snew_ref):
    y = _normalize_block(p0_ref[0] + p1_ref[0])
    y_ref[...] = y
    snew_ref[...] = s_ref[...] + y


_P_SPECS = [
    pl.BlockSpec((1, BR, DIM), lambda i: (0, i, 0)),
    pl.BlockSpec((1, BR, DIM), lambda i: (1, i, 0)),
    pl.BlockSpec((BR, DIM), lambda i: (i, 0)),
]

_combine = pl.pallas_call(
    _combine_body,
    grid=(N // BR,),
    in_specs=_P_SPECS,
    out_specs=[
        pl.BlockSpec((BR, DIM), lambda i: (i, 0)),
        pl.BlockSpec((BR, DIM), lambda i: (i, 0)),
    ],
    out_shape=[
        jax.ShapeDtypeStruct((N, DIM), jnp.float32),
        jax.ShapeDtypeStruct((N, DIM), jnp.float32),
    ],
)


def _combine_final_body(p0_ref, p1_ref, s_ref, out_ref):
    y = _normalize_block(p0_ref[0] + p1_ref[0])
    out_ref[...] = _normalize_block(s_ref[...] + y)


_combine_final = pl.pallas_call(
    _combine_final_body,
    grid=(N // BR,),
    in_specs=_P_SPECS,
    out_specs=pl.BlockSpec((BR, DIM), lambda i: (i, 0)),
    out_shape=jax.ShapeDtypeStruct((N, DIM), jnp.float32),
)


def kernel(edge_index, edge_weight, emb_weight):
    src = edge_index[0]
    dst = edge_index[1]
    pad = EP - E
    srcp = jnp.concatenate([src, jnp.zeros((pad,), jnp.int32)])
    dstp = jnp.concatenate([dst, jnp.zeros((pad,), jnp.int32)])
    wp = jnp.concatenate([edge_weight, jnp.zeros((pad,), jnp.float32)])
    sd_packed = (srcp | (dstp << 16)).reshape(NW, CPW, CHUNK)
    w_packed = wp.reshape(NW, EPW)
    zeros = jnp.zeros((NP, DIM), jnp.float32)

    x = emb_weight
    s = emb_weight
    for layer in range(N_LAYERS):
        partials = _get_sc_layer()(x, sd_packed, w_packed, zeros)
        if layer < N_LAYERS - 1:
            x, s = _combine(partials, partials, s)
        else:
            out = _combine_final(partials, partials, s)
    return out
